# baseline (speedup 1.0000x reference)
import functools

import jax
import jax.numpy as jnp
from jax import lax
from jax.experimental import pallas as pl
from jax.experimental.pallas import tpu as pltpu
from jax.experimental.pallas import tpu_sc as plsc

N_NODES = 100000
EMB_DIM = 128
N_TYPES = 119
CHUNK = 256
NC, NS = 2, 16
NW = NC * NS
NCHUNKS = (N_NODES + CHUNK - 1) // CHUNK
LAST = NCHUNKS - 1
TAIL_ROWS = N_NODES - LAST * CHUNK
SLOTS = (NCHUNKS + NW - 1) // NW
IDX_PER_W = SLOTS * CHUNK
IDX_PAD = 4096
NBUF = 3


@functools.partial(
    pl.kernel,
    out_type=jax.ShapeDtypeStruct((N_NODES, EMB_DIM), jnp.float32),
    mesh=plsc.VectorSubcoreMesh(
        core_axis_name="c", subcore_axis_name="s", num_cores=NC, num_subcores=NS
    ),
    scratch_types=[
        pltpu.VMEM((1, IDX_PAD), jnp.int32),
        pltpu.VMEM((NBUF, CHUNK, EMB_DIM), jnp.float32),
        pltpu.SemaphoreType.DMA,
        pltpu.SemaphoreType.DMA,
    ],
)
def _sc_gather(idx_hbm, table_hbm, out_hbm, idx_v, buf_v, gsem, ssem):
    c = lax.axis_index("c")
    s = lax.axis_index("s")
    w = s * NC + c

    pltpu.sync_copy(idx_hbm.at[w], idx_v)

    def g_desc(j):
        return pltpu.make_async_copy(
            table_hbm.at[idx_v.at[0, pl.ds(CHUNK * j, CHUNK)]],
            buf_v.at[j % NBUF],
            gsem,
        )

    def s_full(j):
        return pltpu.make_async_copy(
            buf_v.at[j % NBUF],
            out_hbm.at[pl.ds((SLOTS * w + j) * CHUNK, CHUNK)],
            ssem,
        )

    def s_tail(j):
        return pltpu.make_async_copy(
            buf_v.at[j % NBUF, pl.ds(0, TAIL_ROWS)],
            out_hbm.at[pl.ds(LAST * CHUNK, TAIL_ROWS)],
            ssem,
        )

    def when_valid(j, fn):
        pl.when(SLOTS * w + j <= LAST)(fn)

    def start_gather(j):
        when_valid(j, lambda: g_desc(j).start())

    def wait_gather(j):
        when_valid(j, lambda: g_desc(j).wait())

    def start_store(j):
        pl.when(SLOTS * w + j < LAST)(lambda: s_full(j).start())
        pl.when(SLOTS * w + j == LAST)(lambda: s_tail(j).start())

    def wait_store(j):
        pl.when(SLOTS * w + j < LAST)(lambda: s_full(j).wait())
        pl.when(SLOTS * w + j == LAST)(lambda: s_tail(j).wait())

    for j in range(NBUF - 1):
        start_gather(j)

    for j in range(SLOTS):
        wait_gather(j)
        start_store(j)
        jn = j + NBUF - 1
        if jn < SLOTS:
            if j >= 1:
                wait_store(j - 1)
            start_gather(jn)

    for j in range(max(0, SLOTS - NBUF), SLOTS):
        wait_store(j)


def kernel(node_feature, atom_type_embedding):
    idx = node_feature[:, 0]
    idx = jnp.pad(idx, (0, NW * IDX_PER_W - N_NODES))
    idx = idx.reshape(NW, IDX_PER_W)
    idx = idx + (jnp.arange(NW, dtype=jnp.int32) * N_TYPES)[:, None]
    idx = jnp.pad(idx, ((0, 0), (0, IDX_PAD - IDX_PER_W)))
    idx = idx.reshape(NW, 1, IDX_PAD)
    table_rep = jnp.tile(atom_type_embedding, (NW, 1))
    return _sc_gather(idx, table_rep)


# speedup vs baseline: 1.0062x; 1.0062x over previous
import functools

import jax
import jax.numpy as jnp
from jax import lax
from jax.experimental import pallas as pl
from jax.experimental.pallas import tpu as pltpu
from jax.experimental.pallas import tpu_sc as plsc

N_NODES = 100000
EMB_DIM = 128
N_TYPES = 119
CHUNK = 128
NC, NS = 2, 16
NW = NC * NS
NCHUNKS = (N_NODES + CHUNK - 1) // CHUNK
LAST = NCHUNKS - 1
TAIL_ROWS = N_NODES - LAST * CHUNK
SLOTS = (NCHUNKS + NW - 1) // NW
IDX_PER_W = SLOTS * CHUNK
IDX_PAD = 4096
NBUF = 6


@functools.partial(
    pl.kernel,
    out_type=jax.ShapeDtypeStruct((N_NODES, EMB_DIM), jnp.float32),
    mesh=plsc.VectorSubcoreMesh(
        core_axis_name="c", subcore_axis_name="s", num_cores=NC, num_subcores=NS
    ),
    compiler_params=pltpu.CompilerParams(skip_device_barrier=True),
    scratch_types=[
        pltpu.VMEM((1, IDX_PAD), jnp.int32),
        pltpu.VMEM((NBUF, CHUNK, EMB_DIM), jnp.float32),
        pltpu.SemaphoreType.DMA,
        pltpu.SemaphoreType.DMA,
    ],
)
def _sc_gather(idx_hbm, table_hbm, out_hbm, idx_v, buf_v, gsem, ssem):
    c = lax.axis_index("c")
    s = lax.axis_index("s")
    w = s * NC + c

    pltpu.sync_copy(idx_hbm.at[w], idx_v)

    def g_desc(j):
        return pltpu.make_async_copy(
            table_hbm.at[idx_v.at[0, pl.ds(CHUNK * j, CHUNK)]],
            buf_v.at[j % NBUF],
            gsem,
        )

    def s_full(j):
        return pltpu.make_async_copy(
            buf_v.at[j % NBUF],
            out_hbm.at[pl.ds((SLOTS * w + j) * CHUNK, CHUNK)],
            ssem,
        )

    def s_tail(j):
        return pltpu.make_async_copy(
            buf_v.at[j % NBUF, pl.ds(0, TAIL_ROWS)],
            out_hbm.at[pl.ds(LAST * CHUNK, TAIL_ROWS)],
            ssem,
        )

    def when_valid(j, fn):
        pl.when(SLOTS * w + j <= LAST)(fn)

    def start_gather(j):
        when_valid(j, lambda: g_desc(j).start())

    def wait_gather(j):
        when_valid(j, lambda: g_desc(j).wait())

    def start_store(j):
        pl.when(SLOTS * w + j < LAST)(lambda: s_full(j).start())
        pl.when(SLOTS * w + j == LAST)(lambda: s_tail(j).start())

    def wait_store(j):
        pl.when(SLOTS * w + j < LAST)(lambda: s_full(j).wait())
        pl.when(SLOTS * w + j == LAST)(lambda: s_tail(j).wait())

    for j in range(NBUF - 1):
        start_gather(j)

    for j in range(SLOTS):
        wait_gather(j)
        start_store(j)
        jn = j + NBUF - 1
        if jn < SLOTS:
            if j >= 1:
                wait_store(j - 1)
            start_gather(jn)

    for j in range(max(0, SLOTS - NBUF), SLOTS):
        wait_store(j)


def kernel(node_feature, atom_type_embedding):
    idx = node_feature[:, 0]
    idx = jnp.pad(idx, (0, NW * IDX_PER_W - N_NODES))
    idx = idx.reshape(NW, IDX_PER_W)
    idx = idx + (jnp.arange(NW, dtype=jnp.int32) * N_TYPES)[:, None]
    idx = jnp.pad(idx, ((0, 0), (0, IDX_PAD - IDX_PER_W)))
    idx = idx.reshape(NW, 1, IDX_PAD)
    table_rep = jnp.tile(atom_type_embedding, (NW, 1))
    return _sc_gather(idx, table_rep)


# speedup vs baseline: 1.0105x; 1.0043x over previous
import functools

import jax
import jax.numpy as jnp
from jax import lax
from jax.experimental import pallas as pl
from jax.experimental.pallas import tpu as pltpu
from jax.experimental.pallas import tpu_sc as plsc

N_NODES = 100000
EMB_DIM = 128
N_TYPES = 119
CHUNK = 128
NC, NS = 2, 16
NW = NC * NS
NCHUNKS = (N_NODES + CHUNK - 1) // CHUNK
LAST = NCHUNKS - 1
TAIL_ROWS = N_NODES - LAST * CHUNK
SLOTS = (NCHUNKS + NW - 1) // NW
IDX_PER_W = SLOTS * CHUNK
IDX_PAD = 4096
NBUF = 6


@functools.partial(
    pl.kernel,
    out_type=jax.ShapeDtypeStruct((N_NODES, EMB_DIM), jnp.float32),
    mesh=plsc.VectorSubcoreMesh(
        core_axis_name="c", subcore_axis_name="s", num_cores=NC, num_subcores=NS
    ),
    scratch_types=[
        pltpu.VMEM((1, IDX_PAD), jnp.int32),
        pltpu.VMEM((NBUF, CHUNK, EMB_DIM), jnp.float32),
        pltpu.SemaphoreType.DMA,
        pltpu.SemaphoreType.DMA,
    ],
)
def _sc_gather(idx_hbm, table_hbm, out_hbm, idx_v, buf_v, gsem, ssem):
    c = lax.axis_index("c")
    s = lax.axis_index("s")
    w = s * NC + c

    pltpu.sync_copy(idx_hbm.at[w], idx_v)

    def g_desc(j):
        return pltpu.make_async_copy(
            table_hbm.at[idx_v.at[0, pl.ds(CHUNK * j, CHUNK)]],
            buf_v.at[j % NBUF],
            gsem,
        )

    def s_full(j):
        return pltpu.make_async_copy(
            buf_v.at[j % NBUF],
            out_hbm.at[pl.ds((SLOTS * w + j) * CHUNK, CHUNK)],
            ssem,
        )

    def s_tail(j):
        return pltpu.make_async_copy(
            buf_v.at[j % NBUF, pl.ds(0, TAIL_ROWS)],
            out_hbm.at[pl.ds(LAST * CHUNK, TAIL_ROWS)],
            ssem,
        )

    def when_valid(j, fn):
        pl.when(SLOTS * w + j <= LAST)(fn)

    def start_gather(j):
        when_valid(j, lambda: g_desc(j).start())

    def wait_gather(j):
        when_valid(j, lambda: g_desc(j).wait())

    def start_store(j):
        pl.when(SLOTS * w + j < LAST)(lambda: s_full(j).start())
        pl.when(SLOTS * w + j == LAST)(lambda: s_tail(j).start())

    def wait_store(j):
        pl.when(SLOTS * w + j < LAST)(lambda: s_full(j).wait())
        pl.when(SLOTS * w + j == LAST)(lambda: s_tail(j).wait())

    for j in range(NBUF - 1):
        start_gather(j)

    for j in range(SLOTS):
        wait_gather(j)
        start_store(j)
        jn = j + NBUF - 1
        if jn < SLOTS:
            if j >= 1:
                wait_store(j - 1)
            start_gather(jn)

    for j in range(max(0, SLOTS - NBUF), SLOTS):
        wait_store(j)


def kernel(node_feature, atom_type_embedding):
    idx = node_feature[:, 0]
    idx = jnp.pad(idx, (0, NW * IDX_PER_W - N_NODES))
    idx = idx.reshape(NW, IDX_PER_W)
    idx = idx + (jnp.arange(NW, dtype=jnp.int32) * N_TYPES)[:, None]
    idx = jnp.pad(idx, ((0, 0), (0, IDX_PAD - IDX_PER_W)))
    idx = idx.reshape(NW, 1, IDX_PAD)
    table_rep = jnp.tile(atom_type_embedding, (NW, 1))
    return _sc_gather(idx, table_rep)
